# Initial kernel scaffold; baseline (speedup 1.0000x reference)
import jax
import jax.numpy as jnp
from jax.experimental import pallas as pl


def kernel(input_matrix, adjacency_coo_matrix, W):
    raise NotImplementedError("write your pallas kernel here")


# speedup vs baseline: 4.8498x; 4.8498x over previous
import functools

import jax
import jax.numpy as jnp
from jax import lax
from jax.experimental import pallas as pl
from jax.experimental.pallas import tpu as pltpu
from jax.experimental.pallas import tpu_sc as plsc

N_NODES = 10000
N_EDGES = 320000
D = 128

NC = 2
NS = 16
N_PAD = 10240

EPT = N_EDGES // NS
CHUNK = 80
NCHUNK = EPT // CHUNK
ROWS_PER_TILE = N_PAD // NS


def _sc_segment_sums(x, src, dst):
    mesh = plsc.VectorSubcoreMesh(core_axis_name="c", subcore_axis_name="s")

    @functools.partial(
        pl.kernel,
        mesh=mesh,
        out_type=jax.ShapeDtypeStruct((NC * N_PAD, D), jnp.float32),
        scratch_types=[
            pltpu.VMEM((CHUNK,), jnp.int32),
            pltpu.VMEM((CHUNK,), jnp.int32),
            pltpu.VMEM((CHUNK, D), jnp.float32),
            pltpu.VMEM_SHARED((N_PAD, D), jnp.float32),
            pltpu.SemaphoreType.DMA,
        ],
    )
    def k(x_hbm, src_hbm, dst_hbm, out_hbm,
          src_v, dst_v, rows_v, acc_sh, sem):
        cid = lax.axis_index("c")
        sid = lax.axis_index("s")
        row0 = sid * ROWS_PER_TILE

        def fill(val):
            v16 = jnp.full((16,), val, jnp.float32)

            def frow(r, carry):
                for j in range(D // 16):
                    rows_v[r, pl.ds(j * 16, 16)] = v16
                return carry

            lax.fori_loop(0, CHUNK, frow, 0)

        fill(0.0)

        def zacc(t, carry):
            pltpu.sync_copy(
                rows_v, acc_sh.at[pl.ds(row0 + t * CHUNK, CHUNK)])
            return carry

        lax.fori_loop(0, ROWS_PER_TILE // CHUNK, zacc, 0)
        fill(1.0)
        plsc.subcore_barrier()

        base = sid * EPT

        def body(i, carry):
            off = base + i * CHUNK
            pltpu.sync_copy(src_hbm.at[pl.ds(off, CHUNK)], src_v)

            @pl.when(cid == 0)
            def _():
                pltpu.sync_copy(dst_hbm.at[pl.ds(off, CHUNK)], dst_v)
                pltpu.async_copy(x_hbm.at[dst_v], rows_v, sem).wait()

            pltpu.sync_copy(rows_v, acc_sh.at[src_v], add=True)
            return carry

        lax.fori_loop(0, NCHUNK, body, 0)
        plsc.subcore_barrier()

        pltpu.sync_copy(
            acc_sh.at[pl.ds(row0, ROWS_PER_TILE)],
            out_hbm.at[pl.ds(cid * N_PAD + row0, ROWS_PER_TILE)],
        )

    return k(x, src, dst)


def _tc_finish(parts, x, w):
    BR = 1024
    grid = N_PAD // BR

    def body(parts_ref, x_ref, w_ref, o_ref):
        s = parts_ref[0] + x_ref[...]
        c = parts_ref[1][:, :1] + 1.0
        acc = jnp.dot(s, w_ref[...], preferred_element_type=jnp.float32)
        o_ref[...] = acc / c

    return pl.pallas_call(
        body,
        grid=(grid,),
        in_specs=[
            pl.BlockSpec((NC, BR, D), lambda i: (0, i, 0)),
            pl.BlockSpec((BR, D), lambda i: (i, 0)),
            pl.BlockSpec((D, D), lambda i: (0, 0)),
        ],
        out_specs=pl.BlockSpec((BR, D), lambda i: (i, 0)),
        out_shape=jax.ShapeDtypeStruct((N_PAD, D), jnp.float32),
    )(parts, x, w)


def kernel(input_matrix, adjacency_coo_matrix, W):
    x = input_matrix
    src = adjacency_coo_matrix[0]
    dst = adjacency_coo_matrix[1]
    raw = _sc_segment_sums(x, src, dst)
    parts = raw.reshape(NC, N_PAD, D)
    xp = jnp.pad(x, ((0, N_PAD - N_NODES), (0, 0)))
    out = _tc_finish(parts, xp, W)
    return out[:N_NODES]


# speedup vs baseline: 10.0396x; 2.0701x over previous
import functools

import jax
import jax.numpy as jnp
from jax import lax
from jax.experimental import pallas as pl
from jax.experimental.pallas import tpu as pltpu
from jax.experimental.pallas import tpu_sc as plsc

N_NODES = 10000
N_EDGES = 320000
D = 128

NC = 2
NS = 16
N_PAD = 10240

EPT = N_EDGES // NS
CHUNK = 40
NCHUNK = EPT // CHUNK
NBUF = 4
ROWS_PER_TILE = N_PAD // NS


def _sc_segment_sums(x, sdx):
    mesh = plsc.VectorSubcoreMesh(core_axis_name="c", subcore_axis_name="s")

    @functools.partial(
        pl.kernel,
        mesh=mesh,
        out_type=jax.ShapeDtypeStruct((NC * N_PAD, D), jnp.float32),
        scratch_types=[
            pltpu.VMEM((NBUF, 2, CHUNK), jnp.int32),
            pltpu.VMEM((NBUF, CHUNK, D), jnp.float32),
            pltpu.VMEM_SHARED((N_PAD, D), jnp.float32),
        ]
        + [pltpu.SemaphoreType.DMA] * (3 * NBUF),
    )
    def k(x_hbm, sdx_hbm, out_hbm, sdx_v, rows_v, acc_sh, *sems):
        isem = sems[:NBUF]
        gsem = sems[NBUF:2 * NBUF]
        ssem = sems[2 * NBUF:]
        cid = lax.axis_index("c")
        sid = lax.axis_index("s")
        row0 = sid * ROWS_PER_TILE

        def fill(val):
            v16 = jnp.full((16,), val, jnp.float32)

            def frow(r, carry):
                for j in range(D // 16):
                    rows_v[0, r, pl.ds(j * 16, 16)] = v16
                return carry

            lax.fori_loop(0, CHUNK, frow, 0)

        fill(0.0)

        def zacc(t, carry):
            pltpu.sync_copy(
                rows_v.at[0], acc_sh.at[pl.ds(row0 + t * CHUNK, CHUNK)])
            return carry

        lax.fori_loop(0, ROWS_PER_TILE // CHUNK, zacc, 0)
        fill(1.0)
        plsc.subcore_barrier()

        def ld_start(b, i):
            pltpu.async_copy(sdx_hbm.at[sid, i], sdx_v.at[b], isem[b])

        def ld_wait(b, i):
            pltpu.make_async_copy(
                sdx_hbm.at[sid, i], sdx_v.at[b], isem[b]).wait()

        def g_start(b, i):
            pltpu.async_copy(
                x_hbm.at[sdx_v.at[b, 1]], rows_v.at[b], gsem[b])

        def g_wait(b, i):
            pltpu.make_async_copy(
                x_hbm.at[sdx_v.at[b, 1]], rows_v.at[b], gsem[b]).wait()

        def s_start(b, i, slot):
            pltpu.async_copy(
                rows_v.at[slot], acc_sh.at[sdx_v.at[b, 0]], ssem[b],
                add=True)

        def s_wait(b, i, slot):
            pltpu.make_async_copy(
                rows_v.at[slot], acc_sh.at[sdx_v.at[b, 0]], ssem[b]).wait()

        M = NBUF

        @pl.when(cid == 0)
        def _():
            for i in range(NBUF):
                ld_start(i % M, i)
            for i in range(3):
                ld_wait(i % M, i)
                g_start(i % M, i)
            for i in range(2):
                g_wait(i % M, i)
                s_start(i % M, i, i % M)

            def step(c0, j):
                c = c0 + j

                def sl(k):
                    return (2 + j + k) % M

                s_wait(sl(-2), c - 2, sl(-2))
                ld_start(sl(2), c + 2)
                ld_wait(sl(1), c + 1)
                g_start(sl(1), c + 1)
                g_wait(sl(0), c)
                s_start(sl(0), c, sl(0))

            def body(p, carry):
                for j in range(NBUF):
                    step(2 + p * NBUF, j)
                return carry

            lax.fori_loop(0, (NCHUNK - 4) // NBUF, body, 0)

            n = NCHUNK
            s_wait((n - 4) % M, n - 4, (n - 4) % M)
            ld_wait((n - 1) % M, n - 1)
            g_start((n - 1) % M, n - 1)
            g_wait((n - 2) % M, n - 2)
            s_start((n - 2) % M, n - 2, (n - 2) % M)
            s_wait((n - 3) % M, n - 3, (n - 3) % M)
            g_wait((n - 1) % M, n - 1)
            s_start((n - 1) % M, n - 1, (n - 1) % M)
            s_wait((n - 2) % M, n - 2, (n - 2) % M)
            s_wait((n - 1) % M, n - 1, (n - 1) % M)

        @pl.when(cid == 1)
        def _():
            for i in range(NBUF):
                ld_start(i % M, i)
            for i in range(2):
                ld_wait(i % M, i)
                s_start(i % M, i, 0)

            def step(c0, j):
                c = c0 + j

                def sl(k):
                    return (2 + j + k) % M

                s_wait(sl(-2), c - 2, 0)
                ld_start(sl(2), c + 2)
                ld_wait(sl(0), c)
                s_start(sl(0), c, 0)

            def body(p, carry):
                for j in range(NBUF):
                    step(2 + p * NBUF, j)
                return carry

            lax.fori_loop(0, (NCHUNK - 4) // NBUF, body, 0)

            n = NCHUNK
            s_wait((n - 4) % M, n - 4, 0)
            ld_wait((n - 2) % M, n - 2)
            s_start((n - 2) % M, n - 2, 0)
            s_wait((n - 3) % M, n - 3, 0)
            ld_wait((n - 1) % M, n - 1)
            s_start((n - 1) % M, n - 1, 0)
            s_wait((n - 2) % M, n - 2, 0)
            s_wait((n - 1) % M, n - 1, 0)

        plsc.subcore_barrier()
        pltpu.sync_copy(
            acc_sh.at[pl.ds(row0, ROWS_PER_TILE)],
            out_hbm.at[pl.ds(cid * N_PAD + row0, ROWS_PER_TILE)],
        )

    return k(x, sdx)


def _tc_finish(parts, x, w):
    BR = 1024
    grid = N_PAD // BR

    def body(parts_ref, x_ref, w_ref, o_ref):
        s = parts_ref[0] + x_ref[...]
        c = parts_ref[1][:, :1] + 1.0
        acc = jnp.dot(s, w_ref[...], preferred_element_type=jnp.float32)
        o_ref[...] = acc / c

    return pl.pallas_call(
        body,
        grid=(grid,),
        in_specs=[
            pl.BlockSpec((NC, BR, D), lambda i: (0, i, 0)),
            pl.BlockSpec((BR, D), lambda i: (i, 0)),
            pl.BlockSpec((D, D), lambda i: (0, 0)),
        ],
        out_specs=pl.BlockSpec((BR, D), lambda i: (i, 0)),
        out_shape=jax.ShapeDtypeStruct((N_PAD, D), jnp.float32),
    )(parts, x, w)


def kernel(input_matrix, adjacency_coo_matrix, W):
    x = input_matrix
    sdx = adjacency_coo_matrix.reshape(
        2, NS, NCHUNK, CHUNK).transpose(1, 2, 0, 3)
    raw = _sc_segment_sums(x, sdx)
    parts = raw.reshape(NC, N_PAD, D)
    xp = jnp.pad(x, ((0, N_PAD - N_NODES), (0, 0)))
    out = _tc_finish(parts, xp, W)
    return out[:N_NODES]


# speedup vs baseline: 14.9016x; 1.4843x over previous
import functools

import jax
import jax.numpy as jnp
from jax import lax
from jax.experimental import pallas as pl
from jax.experimental.pallas import tpu as pltpu
from jax.experimental.pallas import tpu_sc as plsc

N_NODES = 10000
N_EDGES = 320000
D = 128

NC = 2
NS = 16
N_PAD = 10240

EPT = N_EDGES // (NC * NS)
CHUNK = 40
NCHUNK = EPT // CHUNK
M = 4
ROWS_PER_TILE = N_PAD // NS

S_LO = 2
S_HI = 246


def _sc_segment_sums(x, sdx):
    mesh = plsc.VectorSubcoreMesh(core_axis_name="c", subcore_axis_name="s")

    @functools.partial(
        pl.kernel,
        mesh=mesh,
        out_type=[
            jax.ShapeDtypeStruct((NC * N_PAD, D), jnp.float32),
            jax.ShapeDtypeStruct((NC * N_PAD,), jnp.float32),
        ],
        scratch_types=[
            pltpu.VMEM((M, 2, CHUNK), jnp.int32),
            pltpu.VMEM((M, CHUNK, D), jnp.float32),
            pltpu.VMEM((CHUNK,), jnp.float32),
            pltpu.VMEM((ROWS_PER_TILE,), jnp.float32),
            pltpu.VMEM_SHARED((N_PAD, D), jnp.float32),
            pltpu.VMEM_SHARED((N_PAD,), jnp.float32),
        ]
        + [pltpu.SemaphoreType.DMA] * (4 * M),
    )
    def k(x_hbm, sdx_hbm, out_hbm, outc_hbm,
          sdx_v, rows_v, ones_v, zb_v, acc_sh, cacc_sh, *sems):
        isem = sems[:M]
        gsem = sems[M:2 * M]
        ssem = sems[2 * M:3 * M]
        csem = sems[3 * M:]
        cid = lax.axis_index("c")
        sid = lax.axis_index("s")
        row0 = sid * ROWS_PER_TILE

        z16 = jnp.zeros((16,), jnp.float32)
        o16 = jnp.ones((16,), jnp.float32)

        def zrow(r, carry):
            for j in range(D // 16):
                rows_v[0, r, pl.ds(j * 16, 16)] = z16
            return carry

        lax.fori_loop(0, CHUNK, zrow, 0)

        def zacc(t, carry):
            pltpu.sync_copy(
                rows_v.at[0], acc_sh.at[pl.ds(row0 + t * CHUNK, CHUNK)])
            return carry

        lax.fori_loop(0, ROWS_PER_TILE // CHUNK, zacc, 0)

        def zb(t, carry):
            zb_v[pl.ds(t * 16, 16)] = z16
            return carry

        lax.fori_loop(0, ROWS_PER_TILE // 16, zb, 0)
        pltpu.sync_copy(zb_v, cacc_sh.at[pl.ds(row0, ROWS_PER_TILE)])
        ones_v[pl.ds(0, 16)] = o16
        ones_v[pl.ds(16, 16)] = o16
        ones_v[pl.ds(CHUNK - 16, 16)] = o16
        plsc.subcore_barrier()

        def ld_start(si, i):
            pltpu.async_copy(sdx_hbm.at[cid, sid, i], sdx_v.at[si],
                             isem[si])

        def ld_wait(si, i):
            pltpu.make_async_copy(
                sdx_hbm.at[cid, sid, i], sdx_v.at[si], isem[si]).wait()

        def g_start(b, i):
            pltpu.async_copy(
                x_hbm.at[sdx_v.at[b, 1]], rows_v.at[b], gsem[b])

        def g_wait(b, i):
            pltpu.make_async_copy(
                x_hbm.at[sdx_v.at[b, 1]], rows_v.at[b], gsem[b]).wait()

        def s_start(b, i):
            pltpu.async_copy(
                rows_v.at[b], acc_sh.at[sdx_v.at[b, 0]], ssem[b],
                add=True)

        def s_wait(b, i):
            pltpu.make_async_copy(
                rows_v.at[b], acc_sh.at[sdx_v.at[b, 0]], ssem[b]).wait()

        def c_start(b, i):
            pltpu.async_copy(
                ones_v, cacc_sh.at[sdx_v.at[b, 0]], csem[b], add=True)

        def c_wait(b, i):
            pltpu.make_async_copy(
                ones_v, cacc_sh.at[sdx_v.at[b, 0]], csem[b]).wait()

        n = NCHUNK

        def emit(c):
            if 0 <= c - 2:
                s_wait((c - 2) % M, c - 2)
                c_wait((c - 2) % M, c - 2)
            if c + 2 < n:
                ld_start((c + 2) % M, c + 2)
            if 0 <= c + 1 < n:
                ld_wait((c + 1) % M, c + 1)
                g_start((c + 1) % M, c + 1)
            if 0 <= c < n:
                g_wait(c % M, c)
                s_start(c % M, c)
                c_start(c % M, c)

        for c in range(-2, S_LO):
            emit(c)

        def body(p, carry):
            c0 = S_LO + 4 * p
            for j in range(4):
                c = c0 + j
                b0 = (S_LO + j) % M
                b1 = (S_LO + j + 1) % M
                b2 = (S_LO + j + 2) % M

                s_wait(b2, c - 2)
                c_wait(b2, c - 2)
                ld_start(b2, c + 2)
                ld_wait(b1, c + 1)
                g_start(b1, c + 1)
                g_wait(b0, c)
                s_start(b0, c)
                c_start(b0, c)
            return carry

        lax.fori_loop(0, (S_HI - S_LO) // 4, body, 0)
        for c in range(S_HI, n + 2):
            emit(c)

        plsc.subcore_barrier()
        pltpu.sync_copy(
            acc_sh.at[pl.ds(row0, ROWS_PER_TILE)],
            out_hbm.at[pl.ds(cid * N_PAD + row0, ROWS_PER_TILE)],
        )
        pltpu.sync_copy(
            cacc_sh.at[pl.ds(row0, ROWS_PER_TILE)],
            outc_hbm.at[pl.ds(cid * N_PAD + row0, ROWS_PER_TILE)],
        )

    return k(x, sdx)


def _tc_finish(parts, cnt, x, w):
    BR = 1024
    grid = N_PAD // BR

    def body(parts_ref, cnt_ref, x_ref, w_ref, o_ref):
        s = parts_ref[0] + parts_ref[1] + x_ref[...]
        c = cnt_ref[0] + cnt_ref[1] + 1.0
        acc = jnp.dot(s, w_ref[...], preferred_element_type=jnp.float32)
        o_ref[...] = acc / c

    return pl.pallas_call(
        body,
        grid=(grid,),
        in_specs=[
            pl.BlockSpec((NC, BR, D), lambda i: (0, i, 0)),
            pl.BlockSpec((NC, BR, 1), lambda i: (0, i, 0)),
            pl.BlockSpec((BR, D), lambda i: (i, 0)),
            pl.BlockSpec((D, D), lambda i: (0, 0)),
        ],
        out_specs=pl.BlockSpec((BR, D), lambda i: (i, 0)),
        out_shape=jax.ShapeDtypeStruct((N_PAD, D), jnp.float32),
    )(parts, cnt, x, w)


def kernel(input_matrix, adjacency_coo_matrix, W):
    x = input_matrix
    sdx = adjacency_coo_matrix.reshape(
        2, NC, NS, NCHUNK, CHUNK).transpose(1, 2, 3, 0, 4)
    raw, craw = _sc_segment_sums(x, sdx)
    parts = raw.reshape(NC, N_PAD, D)
    cnt = craw.reshape(NC, N_PAD, 1)
    xp = jnp.pad(x, ((0, N_PAD - N_NODES), (0, 0)))
    out = _tc_finish(parts, cnt, xp, W)
    return out[:N_NODES]
